# baseline (speedup 1.0000x reference)
import jax
import jax.numpy as jnp
from jax.experimental import pallas as pl
from jax.experimental.pallas import tpu as pltpu

N_CLASSES = 1000
BANK = 10
DIM = 2048
BATCH = 4096

N_PAD = 1024
_K_BLK = 256
_N_PREP = DIM // _K_BLK
_ROW_BLK = 512
_N_SCORE = BATCH // _ROW_BLK


def _fused_kernel(mem_ref, x_ref, lbl_ref, idx_ref, acc_ref,
                  m1_ref, m2_ref, m3_ref, norm2_ref):
    j = pl.program_id(0)

    @pl.when(j < _N_PREP)
    def _prep():
        mp = jnp.sum(mem_ref[...], axis=1) / BANK - jnp.float32(1e-6)
        mpt = jnp.concatenate(
            [mp.T, jnp.zeros((_K_BLK, N_PAD - N_CLASSES), jnp.float32)],
            axis=1)
        m1 = mpt.astype(jnp.bfloat16)
        r1 = mpt - m1.astype(jnp.float32)
        m2 = r1.astype(jnp.bfloat16)
        m3 = (r1 - m2.astype(jnp.float32)).astype(jnp.bfloat16)
        base = j * _K_BLK
        m1_ref[pl.ds(base, _K_BLK), :] = m1
        m2_ref[pl.ds(base, _K_BLK), :] = m2
        m3_ref[pl.ds(base, _K_BLK), :] = m3
        part = jnp.sum(mpt * mpt, axis=0).reshape(1, N_PAD)

        @pl.when(j == 0)
        def _():
            norm2_ref[...] = jnp.zeros_like(norm2_ref)

        norm2_ref[...] += part

    @pl.when(j >= _N_PREP)
    def _score():
        x = x_ref[...]
        x1 = x.astype(jnp.bfloat16)
        xr = x - x1.astype(jnp.float32)
        x2 = xr.astype(jnp.bfloat16)
        x3 = (xr - x2.astype(jnp.float32)).astype(jnp.bfloat16)

        def bdot(a, b_ref):
            return jax.lax.dot_general(
                a, b_ref[...],
                dimension_numbers=(((1,), (0,)), ((), ())),
                preferred_element_type=jnp.float32,
            )

        n = _ROW_BLK
        d1 = bdot(jnp.concatenate([x1, x2, x3], axis=0), m1_ref)
        d2 = bdot(jnp.concatenate([x1, x2], axis=0), m2_ref)
        d3 = bdot(x1, m3_ref)
        cross = (d1[2 * n:] + d2[n:]) + d3
        cross += d1[n:2 * n] + d2[:n]
        cross += d1[:n]
        s = norm2_ref[...] - 2.0 * cross

        cols = jax.lax.broadcasted_iota(jnp.int32, s.shape, 1)
        s = jnp.where(cols < N_CLASSES, s, jnp.float32(jnp.inf))
        minval = jnp.min(s, axis=1, keepdims=True)
        fcols = cols.astype(jnp.float32)
        idxf = jnp.min(jnp.where(s == minval, fcols, jnp.float32(N_PAD)),
                       axis=1, keepdims=True)
        idx = idxf.astype(jnp.int32)
        idx_ref[...] = idx
        hits = jnp.sum((idx == lbl_ref[...]).astype(jnp.float32)
                       ).reshape(1, 1)

        @pl.when(j == _N_PREP)
        def _():
            acc_ref[...] = jnp.zeros_like(acc_ref)

        acc_ref[...] += hits

        @pl.when(j == _N_PREP + _N_SCORE - 1)
        def _():
            acc_ref[...] = acc_ref[...] * jnp.float32(1.0 / BATCH)


def kernel(instances, instance_labels, memory):
    labels = instance_labels.astype(jnp.int32)

    def _row(j):
        return jnp.maximum(j - _N_PREP, 0)

    idx, acc = pl.pallas_call(
        _fused_kernel,
        grid=(_N_PREP + _N_SCORE,),
        in_specs=[
            pl.BlockSpec((N_CLASSES, BANK, _K_BLK),
                         lambda j: (0, 0, jnp.minimum(j, _N_PREP - 1))),
            pl.BlockSpec((_ROW_BLK, DIM), lambda j: (_row(j), 0)),
            pl.BlockSpec((_ROW_BLK, 1), lambda j: (_row(j), 0)),
        ],
        out_specs=[
            pl.BlockSpec((_ROW_BLK, 1), lambda j: (_row(j), 0)),
            pl.BlockSpec((1, 1), lambda j: (0, 0)),
        ],
        out_shape=[
            jax.ShapeDtypeStruct((BATCH, 1), jnp.int32),
            jax.ShapeDtypeStruct((1, 1), jnp.float32),
        ],
        scratch_shapes=[
            pltpu.VMEM((DIM, N_PAD), jnp.bfloat16),
            pltpu.VMEM((DIM, N_PAD), jnp.bfloat16),
            pltpu.VMEM((DIM, N_PAD), jnp.bfloat16),
            pltpu.VMEM((1, N_PAD), jnp.float32),
        ],
    )(memory, instances, labels)

    return (idx, acc[0, 0])


# speedup vs baseline: 1.9089x; 1.9089x over previous
import jax
import jax.numpy as jnp
from jax.experimental import pallas as pl
from jax.experimental.pallas import tpu as pltpu

N_CLASSES = 1000
BANK = 10
DIM = 2048
BATCH = 4096

N_PAD = 1024
_K_BLK = 256
_N_PREP = DIM // _K_BLK
_ROW_BLK = 512
_N_SCORE = BATCH // _ROW_BLK


def _fused_kernel(mem_ref, x_ref, lbl_ref, idx_ref, acc_ref,
                  m1_ref, m2_ref, m3_ref, norm2_ref):
    j = pl.program_id(0)

    @pl.when(j < _N_PREP)
    def _prep():
        mp = jnp.sum(mem_ref[...], axis=0) / BANK - jnp.float32(1e-6)
        mpt = jnp.concatenate(
            [mp.T, jnp.zeros((_K_BLK, N_PAD - N_CLASSES), jnp.float32)],
            axis=1)
        m1 = mpt.astype(jnp.bfloat16)
        r1 = mpt - m1.astype(jnp.float32)
        m2 = r1.astype(jnp.bfloat16)
        m3 = (r1 - m2.astype(jnp.float32)).astype(jnp.bfloat16)
        base = j * _K_BLK
        m1_ref[pl.ds(base, _K_BLK), :] = m1
        m2_ref[pl.ds(base, _K_BLK), :] = m2
        m3_ref[pl.ds(base, _K_BLK), :] = m3
        part = jnp.sum(mpt * mpt, axis=0).reshape(1, N_PAD)

        @pl.when(j == 0)
        def _():
            norm2_ref[...] = jnp.zeros_like(norm2_ref)

        norm2_ref[...] += part

    @pl.when(j >= _N_PREP)
    def _score():
        x = x_ref[...]
        x1 = x.astype(jnp.bfloat16)
        xr = x - x1.astype(jnp.float32)
        x2 = xr.astype(jnp.bfloat16)
        x3 = (xr - x2.astype(jnp.float32)).astype(jnp.bfloat16)

        def bdot(a, b_ref):
            return jax.lax.dot_general(
                a, b_ref[...],
                dimension_numbers=(((1,), (0,)), ((), ())),
                preferred_element_type=jnp.float32,
            )

        n = _ROW_BLK
        d1 = bdot(jnp.concatenate([x1, x2, x3], axis=0), m1_ref)
        d2 = bdot(jnp.concatenate([x1, x2], axis=0), m2_ref)
        d3 = bdot(x1, m3_ref)
        cross = (d1[2 * n:] + d2[n:]) + d3
        cross += d1[n:2 * n] + d2[:n]
        cross += d1[:n]
        s = norm2_ref[...] - 2.0 * cross

        cols = jax.lax.broadcasted_iota(jnp.int32, s.shape, 1)
        s = jnp.where(cols < N_CLASSES, s, jnp.float32(jnp.inf))
        minval = jnp.min(s, axis=1, keepdims=True)
        fcols = cols.astype(jnp.float32)
        idxf = jnp.min(jnp.where(s == minval, fcols, jnp.float32(N_PAD)),
                       axis=1, keepdims=True)
        idx = idxf.astype(jnp.int32)
        idx_ref[...] = idx
        hits = jnp.sum((idx == lbl_ref[...]).astype(jnp.float32)
                       ).reshape(1, 1)

        @pl.when(j == _N_PREP)
        def _():
            acc_ref[...] = jnp.zeros_like(acc_ref)

        acc_ref[...] += hits

        @pl.when(j == _N_PREP + _N_SCORE - 1)
        def _():
            acc_ref[...] = acc_ref[...] * jnp.float32(1.0 / BATCH)


def kernel(instances, instance_labels, memory):
    labels = instance_labels.astype(jnp.int32)
    mem_t = jnp.transpose(memory, (1, 0, 2))

    def _row(j):
        return jnp.maximum(j - _N_PREP, 0)

    idx, acc = pl.pallas_call(
        _fused_kernel,
        grid=(_N_PREP + _N_SCORE,),
        in_specs=[
            pl.BlockSpec((BANK, N_CLASSES, _K_BLK),
                         lambda j: (0, 0, jnp.minimum(j, _N_PREP - 1))),
            pl.BlockSpec((_ROW_BLK, DIM), lambda j: (_row(j), 0)),
            pl.BlockSpec((_ROW_BLK, 1), lambda j: (_row(j), 0)),
        ],
        out_specs=[
            pl.BlockSpec((_ROW_BLK, 1), lambda j: (_row(j), 0)),
            pl.BlockSpec((1, 1), lambda j: (0, 0)),
        ],
        out_shape=[
            jax.ShapeDtypeStruct((BATCH, 1), jnp.int32),
            jax.ShapeDtypeStruct((1, 1), jnp.float32),
        ],
        scratch_shapes=[
            pltpu.VMEM((DIM, N_PAD), jnp.bfloat16),
            pltpu.VMEM((DIM, N_PAD), jnp.bfloat16),
            pltpu.VMEM((DIM, N_PAD), jnp.bfloat16),
            pltpu.VMEM((1, N_PAD), jnp.float32),
        ],
    )(mem_t, instances, labels)

    return (idx, acc[0, 0])
